# baseline (speedup 1.0000x reference)
import functools

import jax
import jax.numpy as jnp
from jax.experimental import pallas as pl
from jax.experimental.pallas import tpu as pltpu

NEG_SLOPE = 0.01
BN_EPS = 1e-5

MARGIN = 128
SPAN = 4096


def _leaky(v):
    return jnp.where(v >= 0, v, NEG_SLOPE * v)


def _fused_kernel(x_ref, w1_ref, w2s_ref, wdm_ref, bn_ref, o_ref,
                  xbs_ref, sacc_ref, ssacc_ref, w2e_ref, shift_ref, ys_ref,
                  *, N, H, W, Wp, P, chunks):
    ph = pl.program_id(0)
    i = pl.program_id(1)
    C = 64
    bf16 = jnp.bfloat16

    @pl.when(ph == 0)
    def _phase0():
        x = x_ref[...]

        @pl.when(i == 0)
        def _init():
            sacc_ref[...] = jnp.zeros_like(sacc_ref)
            ssacc_ref[...] = jnp.zeros_like(ssacc_ref)

        d = jnp.dot(wdm_ref[...], x, preferred_element_type=jnp.float32)
        s = jnp.sum(d, axis=1, keepdims=True)
        ss = jnp.sum(d * d, axis=1, keepdims=True)
        sacc_ref[...] = sacc_ref[...] + jnp.broadcast_to(s, sacc_ref.shape)
        ssacc_ref[...] = ssacc_ref[...] + jnp.broadcast_to(ss, ssacc_ref.shape)

        xc = x.astype(bf16)
        xbi = xbs_ref.at[i]
        xbi[...] = jnp.zeros((C, SPAN), bf16)
        for h in range(H):
            lo = MARGIN + (h + 1) * Wp + 1
            xbi[:, lo:lo + W] = xc[:, h * W:(h + 1) * W]

    @pl.when(ph == 1)
    def _phase1():
        @pl.when(i == 0)
        def _finalize_stats():
            inv_cnt = 1.0 / float(N * H * W)
            s = sacc_ref[:, 0:1]
            ss = ssacc_ref[:, 0:1]
            mean = s * inv_cnt
            var = jnp.maximum(ss * inv_cnt - mean * mean, 0.0)
            gamma = bn_ref[:, 1:2]
            beta = bn_ref[:, 2:3]
            b2 = bn_ref[:, 3:4]
            scale = gamma * jax.lax.rsqrt(var + BN_EPS)
            shift_ref[...] = jnp.broadcast_to(
                beta + b2 - mean * scale, shift_ref.shape)
            w2e_ref[...] = jnp.zeros_like(w2e_ref)
            w2e_ref[:, :3 * C] = w2s_ref[...]
            w2e_ref[C:2 * C, 3 * C:] = (wdm_ref[...] * scale).astype(bf16)

        b1 = bn_ref[:, 0:1]
        shift = shift_ref[:, 0:1]
        w1 = w1_ref[...]
        w2e = w2e_ref[...]

        q = jax.lax.broadcasted_iota(jnp.int32, (1, P), 1)
        hh = q >> 6
        ww = q & (Wp - 1)
        interior = (hh >= 1) & (hh <= H) & (ww >= 1) & (ww <= W)
        m = interior.astype(jnp.float32)

        ys_ref[:, 0:MARGIN] = jnp.zeros((C, MARGIN), bf16)
        ys_ref[:, MARGIN + P:] = jnp.zeros((C, SPAN - MARGIN - P), bf16)

        for (lo, q_w) in chunks:
            base = MARGIN + lo
            xw = xbs_ref[i, :, base - 128:base + q_w + 128]
            x3 = jnp.concatenate([xw[:, 63:63 + q_w + 128],
                                  xw[:, 64:64 + q_w + 128],
                                  xw[:, 65:65 + q_w + 128]], axis=0)
            o1 = jnp.dot(w1, x3, preferred_element_type=jnp.float32)
            c1 = (o1[:C, 0:q_w] + o1[C:2 * C, 64:64 + q_w]
                  + o1[2 * C:, 128:128 + q_w])
            yv = (_leaky(c1 + b1) * m[:, lo:lo + q_w]).astype(bf16)
            ys_ref[:, base:base + q_w] = yv

        for (lo, q_w) in chunks:
            base = MARGIN + lo
            yw = ys_ref[:, base - 128:base + q_w + 256]
            xw = xbs_ref[i, :, base - 128:base + q_w + 256]
            x2 = jnp.concatenate([yw[:, 128:128 + q_w + 128],
                                  yw[:, 129:129 + q_w + 128],
                                  yw[:, 130:130 + q_w + 128],
                                  xw[:, 129:129 + q_w + 128]], axis=0)
            o2 = jnp.dot(w2e, x2, preferred_element_type=jnp.float32)
            oc = (o2[:C, 0:q_w] + o2[C:2 * C, 64:64 + q_w]
                  + o2[2 * C:, 128:128 + q_w])
            outv = _leaky(oc + shift)
            h0 = lo // Wp
            for h in range(h0, min(h0 + q_w // Wp, H)):
                o_ref[:, h * W:(h + 1) * W] = \
                    outv[:, (h - h0) * Wp:(h - h0) * Wp + W]


def kernel(x_nchw, w1, b1, w2, b2, wd, bd, gamma, beta):
    del bd
    x_nchw = x_nchw.astype(jnp.float32)
    N, Cin, H, W = x_nchw.shape
    Cout = w1.shape[0]
    f32 = jnp.float32
    bf16 = jnp.bfloat16
    assert Cin == 64 and Cout == 64, "layout assumes 64 channels"

    Hp = H + 2
    Wp = 64
    P = Hp * Wp
    HW = H * W
    chunks = [(0, 1024), (1024, 1024), (2048, 1024), (3072, P - 3072)]

    x_flat = x_nchw.reshape(N, Cin, HW)
    wdm = wd.reshape(Cout, Cin)

    w1s = jnp.transpose(w1, (2, 0, 3, 1)).reshape(3 * Cout, 3 * Cin)
    w2s = jnp.transpose(w2, (2, 0, 3, 1)).reshape(3 * Cout, 3 * Cout)
    bnmat = jnp.stack([b1, gamma, beta, b2], axis=1).astype(f32)

    fused = functools.partial(_fused_kernel, N=N, H=H, W=W, Wp=Wp, P=P,
                              chunks=chunks)
    out_flat = pl.pallas_call(
        fused,
        out_shape=jax.ShapeDtypeStruct((N, Cout, HW), f32),
        grid=(2, N),
        in_specs=[
            pl.BlockSpec((None, Cin, HW),
                         lambda p, i: (jnp.where(p == 0, i, 0), 0, 0)),
            pl.BlockSpec((3 * Cout, 3 * Cin), lambda p, i: (0, 0)),
            pl.BlockSpec((3 * Cout, 3 * Cout), lambda p, i: (0, 0)),
            pl.BlockSpec((Cout, Cin), lambda p, i: (0, 0)),
            pl.BlockSpec((Cout, 4), lambda p, i: (0, 0)),
        ],
        out_specs=pl.BlockSpec((None, Cout, HW),
                               lambda p, i: (jnp.where(p == 1, i, 0), 0, 0)),
        scratch_shapes=[
            pltpu.VMEM((N, Cin, SPAN), bf16),
            pltpu.VMEM((Cout, 128), f32),
            pltpu.VMEM((Cout, 128), f32),
            pltpu.VMEM((3 * Cout, 3 * Cout + Cin), bf16),
            pltpu.VMEM((Cout, 128), f32),
            pltpu.VMEM((Cout, SPAN), bf16),
        ],
        compiler_params=pltpu.CompilerParams(
            dimension_semantics=("arbitrary", "arbitrary"),
            vmem_limit_bytes=64 * 1024 * 1024),
        cost_estimate=pl.CostEstimate(
            flops=2 * N * P * (3 * Cin * 3 * Cout + (3 * Cout + Cin) * 3 * Cout)
            + 2 * N * Cin * Cout * HW,
            transcendentals=0,
            bytes_accessed=4 * N * Cin * HW + 4 * N * Cout * HW),
    )(x_flat, w1s.astype(bf16), w2s.astype(bf16), wdm, bnmat)

    return out_flat.reshape(N, Cout, H, W)


# speedup vs baseline: 1.2752x; 1.2752x over previous
import functools

import jax
import jax.numpy as jnp
from jax.experimental import pallas as pl
from jax.experimental.pallas import tpu as pltpu

NEG_SLOPE = 0.01
BN_EPS = 1e-5


def _leaky(v):
    return jnp.where(v >= 0, v, NEG_SLOPE * v)


def _rot(a, s, p):
    s = s % p
    if s == 0:
        return a
    return jnp.concatenate([a[:, s:], a[:, :s]], axis=1)


def _kw_stack(a, p):
    return jnp.concatenate([_rot(a, -1, p), a, _rot(a, 1, p)], axis=0)


def _fused_kernel(x_ref, w1_ref, w2s_ref, wdm_ref, bn_ref, o_ref,
                  xbs_ref, sacc_ref, ssacc_ref, w2e_ref, shift_ref,
                  *, N, H, W, Wp, P, IPB):
    ph = pl.program_id(0)
    i = pl.program_id(1)
    C = 64
    bf16 = jnp.bfloat16

    @pl.when(ph == 0)
    def _phase0():
        @pl.when(i == 0)
        def _init():
            sacc_ref[...] = jnp.zeros_like(sacc_ref)
            ssacc_ref[...] = jnp.zeros_like(ssacc_ref)

        s_tot = jnp.zeros((C, 1), jnp.float32)
        ss_tot = jnp.zeros((C, 1), jnp.float32)
        for k in range(IPB):
            x = x_ref[k]
            d = jnp.dot(wdm_ref[...], x, preferred_element_type=jnp.float32)
            s_tot = s_tot + jnp.sum(d, axis=1, keepdims=True)
            ss_tot = ss_tot + jnp.sum(d * d, axis=1, keepdims=True)
            xc = x.astype(bf16)
            xbi = xbs_ref.at[i * IPB + k]
            xbi[...] = jnp.zeros((C, P), bf16)
            for h in range(H):
                lo = (h + 1) * Wp + 1
                xbi[:, lo:lo + W] = xc[:, h * W:(h + 1) * W]
        sacc_ref[...] = sacc_ref[...] + jnp.broadcast_to(s_tot, sacc_ref.shape)
        ssacc_ref[...] = ssacc_ref[...] + jnp.broadcast_to(ss_tot,
                                                           ssacc_ref.shape)

    @pl.when(ph == 1)
    def _phase1():
        @pl.when(i == 0)
        def _finalize_stats():
            inv_cnt = 1.0 / float(N * H * W)
            s = sacc_ref[:, 0:1]
            ss = ssacc_ref[:, 0:1]
            mean = s * inv_cnt
            var = jnp.maximum(ss * inv_cnt - mean * mean, 0.0)
            gamma = bn_ref[:, 1:2]
            beta = bn_ref[:, 2:3]
            b2 = bn_ref[:, 3:4]
            scale = gamma * jax.lax.rsqrt(var + BN_EPS)
            shift_ref[...] = jnp.broadcast_to(
                beta + b2 - mean * scale, shift_ref.shape)
            w2e_ref[...] = jnp.zeros_like(w2e_ref)
            w2e_ref[:, :3 * C] = w2s_ref[...]
            w2e_ref[C:2 * C, 3 * C:] = (wdm_ref[...] * scale).astype(bf16)

        b1 = bn_ref[:, 0:1]
        shift = shift_ref[:, 0:1]
        w1 = w1_ref[...]
        w2e = w2e_ref[...]

        q = jax.lax.broadcasted_iota(jnp.int32, (1, P), 1)
        hh = q >> 6
        ww = q & (Wp - 1)
        interior = (hh >= 1) & (hh <= H) & (ww >= 1) & (ww <= W)
        m = interior.astype(jnp.float32)

        for k in range(IPB):
            xb = xbs_ref[i * IPB + k]

            x3 = _kw_stack(xb, P)
            o1 = jnp.dot(w1, x3, preferred_element_type=jnp.float32)
            c1 = (_rot(o1[:C], -Wp, P) + o1[C:2 * C]
                  + _rot(o1[2 * C:], Wp, P))
            y = (_leaky(c1 + b1) * m).astype(bf16)

            y3 = _kw_stack(y, P)
            x2 = jnp.concatenate([y3, xb], axis=0)
            o2 = jnp.dot(w2e, x2, preferred_element_type=jnp.float32)
            o2c = (_rot(o2[:C], 1, P) + _rot(o2[C:2 * C], Wp + 1, P)
                   + _rot(o2[2 * C:], 2 * Wp + 1, P))
            out = _leaky(o2c + shift)
            for h in range(H):
                o_ref[k, :, h * W:(h + 1) * W] = \
                    out[:, h * Wp:h * Wp + W]


def kernel(x_nchw, w1, b1, w2, b2, wd, bd, gamma, beta):
    del bd
    x_nchw = x_nchw.astype(jnp.float32)
    N, Cin, H, W = x_nchw.shape
    Cout = w1.shape[0]
    f32 = jnp.float32
    bf16 = jnp.bfloat16
    assert Cin == 64 and Cout == 64, "layout assumes 64 channels"

    Hp = H + 2
    Wp = 64
    P = Hp * Wp
    HW = H * W
    IPB = 2

    x_flat = x_nchw.reshape(N, Cin, HW)
    wdm = wd.reshape(Cout, Cin)

    w1s = jnp.transpose(w1, (2, 0, 3, 1)).reshape(3 * Cout, 3 * Cin)
    w2s = jnp.transpose(w2, (2, 0, 3, 1)).reshape(3 * Cout, 3 * Cout)
    bnmat = jnp.stack([b1, gamma, beta, b2], axis=1).astype(f32)

    fused = functools.partial(_fused_kernel, N=N, H=H, W=W, Wp=Wp, P=P,
                              IPB=IPB)
    out_flat = pl.pallas_call(
        fused,
        out_shape=jax.ShapeDtypeStruct((N, Cout, HW), f32),
        grid=(2, N // IPB),
        in_specs=[
            pl.BlockSpec((IPB, Cin, HW),
                         lambda p, i: (jnp.where(p == 0, i, 0), 0, 0)),
            pl.BlockSpec((3 * Cout, 3 * Cin), lambda p, i: (0, 0)),
            pl.BlockSpec((3 * Cout, 3 * Cout), lambda p, i: (0, 0)),
            pl.BlockSpec((Cout, Cin), lambda p, i: (0, 0)),
            pl.BlockSpec((Cout, 4), lambda p, i: (0, 0)),
        ],
        out_specs=pl.BlockSpec((IPB, Cout, HW),
                               lambda p, i: (jnp.where(p == 1, i, 0), 0, 0)),
        scratch_shapes=[
            pltpu.VMEM((N, Cin, P), bf16),
            pltpu.VMEM((Cout, 128), f32),
            pltpu.VMEM((Cout, 128), f32),
            pltpu.VMEM((3 * Cout, 3 * Cout + Cin), bf16),
            pltpu.VMEM((Cout, 128), f32),
        ],
        compiler_params=pltpu.CompilerParams(
            dimension_semantics=("arbitrary", "arbitrary"),
            vmem_limit_bytes=64 * 1024 * 1024),
        cost_estimate=pl.CostEstimate(
            flops=2 * N * P * (3 * Cin * 3 * Cout + (3 * Cout + Cin) * 3 * Cout)
            + 2 * N * Cin * Cout * HW,
            transcendentals=0,
            bytes_accessed=4 * N * Cin * HW + 4 * N * Cout * HW),
    )(x_flat, w1s.astype(bf16), w2s.astype(bf16), wdm, bnmat)

    return out_flat.reshape(N, Cout, H, W)


# speedup vs baseline: 1.2771x; 1.0016x over previous
import functools

import jax
import jax.numpy as jnp
from jax.experimental import pallas as pl
from jax.experimental.pallas import tpu as pltpu

NEG_SLOPE = 0.01
BN_EPS = 1e-5


def _leaky(v):
    return jnp.maximum(v, NEG_SLOPE * v)


def _rot(a, s, p):
    s = s % p
    if s == 0:
        return a
    return jnp.concatenate([a[:, s:], a[:, :s]], axis=1)


def _kw_stack(a, p):
    return jnp.concatenate([_rot(a, -1, p), a, _rot(a, 1, p)], axis=0)


def _fused_kernel(x_ref, w1_ref, w2s_ref, wdm_ref, bn_ref, o_ref,
                  xbs_ref, sacc_ref, ssacc_ref, w2e_ref, shift_ref,
                  *, N, H, W, Wp, P, IPB):
    ph = pl.program_id(0)
    i = pl.program_id(1)
    C = 64
    bf16 = jnp.bfloat16

    @pl.when(ph == 0)
    def _phase0():
        @pl.when(i == 0)
        def _init():
            sacc_ref[...] = jnp.zeros_like(sacc_ref)
            ssacc_ref[...] = jnp.zeros_like(ssacc_ref)

        s_tot = jnp.zeros((C, 1), jnp.float32)
        ss_tot = jnp.zeros((C, 1), jnp.float32)
        for k in range(IPB):
            x = x_ref[k]
            d = jnp.dot(wdm_ref[...], x, preferred_element_type=jnp.float32)
            s_tot = s_tot + jnp.sum(d, axis=1, keepdims=True)
            ss_tot = ss_tot + jnp.sum(d * d, axis=1, keepdims=True)
            xc = x.astype(bf16)
            xbi = xbs_ref.at[i * IPB + k]
            xbi[...] = jnp.zeros((C, P), bf16)
            for h in range(H):
                lo = (h + 1) * Wp + 1
                xbi[:, lo:lo + W] = xc[:, h * W:(h + 1) * W]
        sacc_ref[...] = sacc_ref[...] + jnp.broadcast_to(s_tot, sacc_ref.shape)
        ssacc_ref[...] = ssacc_ref[...] + jnp.broadcast_to(ss_tot,
                                                           ssacc_ref.shape)

    @pl.when(ph == 1)
    def _phase1():
        @pl.when(i == 0)
        def _finalize_stats():
            inv_cnt = 1.0 / float(N * H * W)
            s = sacc_ref[:, 0:1]
            ss = ssacc_ref[:, 0:1]
            mean = s * inv_cnt
            var = jnp.maximum(ss * inv_cnt - mean * mean, 0.0)
            gamma = bn_ref[:, 1:2]
            beta = bn_ref[:, 2:3]
            b2 = bn_ref[:, 3:4]
            scale = gamma * jax.lax.rsqrt(var + BN_EPS)
            shift_ref[...] = jnp.broadcast_to(
                beta + b2 - mean * scale, shift_ref.shape)
            w2e_ref[...] = jnp.zeros_like(w2e_ref)
            w2e_ref[:, :3 * C] = w2s_ref[...]
            w2e_ref[C:2 * C, 3 * C:] = (wdm_ref[...] * scale).astype(bf16)

        b1 = bn_ref[:, 0:1]
        shift = shift_ref[:, 0:1]
        w1 = w1_ref[...]
        w2e = w2e_ref[...]

        q = jax.lax.broadcasted_iota(jnp.int32, (1, P), 1)
        hh = q >> 6
        ww = q & (Wp - 1)
        interior = (hh >= 1) & (hh <= H) & (ww >= 1) & (ww <= W)
        m = interior.astype(jnp.float32)

        for k in range(IPB):
            xb = xbs_ref[i * IPB + k]

            x3 = _kw_stack(xb, P)
            o1 = jnp.dot(w1, x3, preferred_element_type=jnp.float32)
            c1 = (_rot(o1[:C], -Wp, P) + o1[C:2 * C]
                  + _rot(o1[2 * C:], Wp, P))
            y = (_leaky(c1 + b1) * m).astype(bf16)

            y3 = _kw_stack(y, P)
            x2 = jnp.concatenate([y3, xb], axis=0)
            o2 = jnp.dot(w2e, x2, preferred_element_type=jnp.float32)
            o2c = (_rot(o2[:C], 1, P) + _rot(o2[C:2 * C], Wp + 1, P)
                   + _rot(o2[2 * C:], 2 * Wp + 1, P))
            out = _leaky(o2c + shift)
            for h in range(H):
                o_ref[k, :, h * W:(h + 1) * W] = \
                    out[:, h * Wp:h * Wp + W]


def kernel(x_nchw, w1, b1, w2, b2, wd, bd, gamma, beta):
    del bd
    x_nchw = x_nchw.astype(jnp.float32)
    N, Cin, H, W = x_nchw.shape
    Cout = w1.shape[0]
    f32 = jnp.float32
    bf16 = jnp.bfloat16
    assert Cin == 64 and Cout == 64, "layout assumes 64 channels"

    Hp = H + 2
    Wp = 64
    P = Hp * Wp
    HW = H * W
    IPB = 4

    x_flat = x_nchw.reshape(N, Cin, HW)
    wdm = wd.reshape(Cout, Cin)

    w1s = jnp.transpose(w1, (2, 0, 3, 1)).reshape(3 * Cout, 3 * Cin)
    w2s = jnp.transpose(w2, (2, 0, 3, 1)).reshape(3 * Cout, 3 * Cout)
    bnmat = jnp.stack([b1, gamma, beta, b2], axis=1).astype(f32)

    fused = functools.partial(_fused_kernel, N=N, H=H, W=W, Wp=Wp, P=P,
                              IPB=IPB)
    out_flat = pl.pallas_call(
        fused,
        out_shape=jax.ShapeDtypeStruct((N, Cout, HW), f32),
        grid=(2, N // IPB),
        in_specs=[
            pl.BlockSpec((IPB, Cin, HW),
                         lambda p, i: (jnp.where(p == 0, i, 0), 0, 0)),
            pl.BlockSpec((3 * Cout, 3 * Cin), lambda p, i: (0, 0)),
            pl.BlockSpec((3 * Cout, 3 * Cout), lambda p, i: (0, 0)),
            pl.BlockSpec((Cout, Cin), lambda p, i: (0, 0)),
            pl.BlockSpec((Cout, 4), lambda p, i: (0, 0)),
        ],
        out_specs=pl.BlockSpec((IPB, Cout, HW),
                               lambda p, i: (jnp.where(p == 1, i, 0), 0, 0)),
        scratch_shapes=[
            pltpu.VMEM((N, Cin, P), bf16),
            pltpu.VMEM((Cout, 128), f32),
            pltpu.VMEM((Cout, 128), f32),
            pltpu.VMEM((3 * Cout, 3 * Cout + Cin), bf16),
            pltpu.VMEM((Cout, 128), f32),
        ],
        compiler_params=pltpu.CompilerParams(
            dimension_semantics=("arbitrary", "arbitrary"),
            vmem_limit_bytes=64 * 1024 * 1024),
        cost_estimate=pl.CostEstimate(
            flops=2 * N * P * (3 * Cin * 3 * Cout + (3 * Cout + Cin) * 3 * Cout)
            + 2 * N * Cin * Cout * HW,
            transcendentals=0,
            bytes_accessed=4 * N * Cin * HW + 4 * N * Cout * HW),
    )(x_flat, w1s.astype(bf16), w2s.astype(bf16), wdm, bnmat)

    return out_flat.reshape(N, Cout, H, W)


# speedup vs baseline: 1.2859x; 1.0068x over previous
import functools

import jax
import jax.numpy as jnp
from jax.experimental import pallas as pl
from jax.experimental.pallas import tpu as pltpu

NEG_SLOPE = 0.01
BN_EPS = 1e-5


def _leaky(v):
    return jnp.maximum(v, NEG_SLOPE * v)


def _rot(a, s, p):
    s = s % p
    if s == 0:
        return a
    return jnp.concatenate([a[:, s:], a[:, :s]], axis=1)


def _kw_stack(a, p):
    return jnp.concatenate([_rot(a, -1, p), a, _rot(a, 1, p)], axis=0)


def _fused_kernel(x_ref, w1_ref, w2s_ref, wdm_ref, bn_ref, o_ref,
                  xbs_ref, sacc_ref, ssacc_ref, w2e_ref, shift_ref,
                  *, N, H, W, Wp, P, IPB):
    ph = pl.program_id(0)
    i = pl.program_id(1)
    C = 64
    bf16 = jnp.bfloat16

    @pl.when(ph == 0)
    def _phase0():
        @pl.when(i == 0)
        def _init():
            sacc_ref[...] = jnp.zeros_like(sacc_ref)
            ssacc_ref[...] = jnp.zeros_like(ssacc_ref)

        s_tot = jnp.zeros((C, 1), jnp.float32)
        ss_tot = jnp.zeros((C, 1), jnp.float32)
        for k in range(IPB):
            x = x_ref[k]
            d = jnp.dot(wdm_ref[...], x, preferred_element_type=jnp.float32)
            s_tot = s_tot + jnp.sum(d, axis=1, keepdims=True)
            ss_tot = ss_tot + jnp.sum(d * d, axis=1, keepdims=True)
            xc = x.astype(bf16)
            xbi = xbs_ref.at[i * IPB + k]
            xbi[...] = jnp.zeros((C, P), bf16)
            for h in range(H):
                lo = (h + 1) * Wp + 1
                xbi[:, lo:lo + W] = xc[:, h * W:(h + 1) * W]
        sacc_ref[...] = sacc_ref[...] + jnp.broadcast_to(s_tot, sacc_ref.shape)
        ssacc_ref[...] = ssacc_ref[...] + jnp.broadcast_to(ss_tot,
                                                           ssacc_ref.shape)

    @pl.when(ph == 1)
    def _phase1():
        @pl.when(i == 0)
        def _finalize_stats():
            inv_cnt = 1.0 / float(N * H * W)
            s = sacc_ref[:, 0:1]
            ss = ssacc_ref[:, 0:1]
            mean = s * inv_cnt
            var = jnp.maximum(ss * inv_cnt - mean * mean, 0.0)
            gamma = bn_ref[:, 1:2]
            beta = bn_ref[:, 2:3]
            b2 = bn_ref[:, 3:4]
            scale = gamma * jax.lax.rsqrt(var + BN_EPS)
            shift_ref[...] = jnp.broadcast_to(
                beta + b2 - mean * scale, shift_ref.shape)
            w2e_ref[...] = jnp.zeros_like(w2e_ref)
            w2e_ref[:, :3 * C] = w2s_ref[...]
            w2e_ref[C:2 * C, 3 * C:] = (wdm_ref[...] * scale).astype(bf16)

        b1 = bn_ref[:, 0:1]
        shift = shift_ref[:, 0:1]
        w1 = w1_ref[...]
        w2e = w2e_ref[...]

        q = jax.lax.broadcasted_iota(jnp.int32, (1, P), 1)
        hh = q >> 6
        ww = q & (Wp - 1)
        interior = (hh >= 1) & (hh <= H) & (ww >= 1) & (ww <= W)
        m = interior.astype(jnp.float32)

        ys = []
        for k in range(IPB):
            xb = xbs_ref[i * IPB + k]
            x3 = _kw_stack(xb, P)
            o1 = jnp.dot(w1, x3, preferred_element_type=jnp.float32)
            c1 = (_rot(o1[:C], -Wp, P) + o1[C:2 * C]
                  + _rot(o1[2 * C:], Wp, P))
            ys.append((_leaky(c1 + b1) * m).astype(bf16))

        for k in range(IPB):
            xb = xbs_ref[i * IPB + k]
            y3 = _kw_stack(ys[k], P)
            x2 = jnp.concatenate([y3, xb], axis=0)
            o2 = jnp.dot(w2e, x2, preferred_element_type=jnp.float32)
            o2c = (_rot(o2[:C], 1, P) + _rot(o2[C:2 * C], Wp + 1, P)
                   + _rot(o2[2 * C:], 2 * Wp + 1, P))
            out = _leaky(o2c + shift)
            for h in range(H):
                o_ref[k, :, h * W:(h + 1) * W] = \
                    out[:, h * Wp:h * Wp + W]


def kernel(x_nchw, w1, b1, w2, b2, wd, bd, gamma, beta):
    del bd
    x_nchw = x_nchw.astype(jnp.float32)
    N, Cin, H, W = x_nchw.shape
    Cout = w1.shape[0]
    f32 = jnp.float32
    bf16 = jnp.bfloat16
    assert Cin == 64 and Cout == 64, "layout assumes 64 channels"

    Hp = H + 2
    Wp = 64
    P = Hp * Wp
    HW = H * W
    IPB = 4

    x_flat = x_nchw.reshape(N, Cin, HW)
    wdm = wd.reshape(Cout, Cin)

    w1s = jnp.transpose(w1, (2, 0, 3, 1)).reshape(3 * Cout, 3 * Cin)
    w2s = jnp.transpose(w2, (2, 0, 3, 1)).reshape(3 * Cout, 3 * Cout)
    bnmat = jnp.stack([b1, gamma, beta, b2], axis=1).astype(f32)

    fused = functools.partial(_fused_kernel, N=N, H=H, W=W, Wp=Wp, P=P,
                              IPB=IPB)
    out_flat = pl.pallas_call(
        fused,
        out_shape=jax.ShapeDtypeStruct((N, Cout, HW), f32),
        grid=(2, N // IPB),
        in_specs=[
            pl.BlockSpec((IPB, Cin, HW),
                         lambda p, i: (jnp.where(p == 0, i, 0), 0, 0)),
            pl.BlockSpec((3 * Cout, 3 * Cin), lambda p, i: (0, 0)),
            pl.BlockSpec((3 * Cout, 3 * Cout), lambda p, i: (0, 0)),
            pl.BlockSpec((Cout, Cin), lambda p, i: (0, 0)),
            pl.BlockSpec((Cout, 4), lambda p, i: (0, 0)),
        ],
        out_specs=pl.BlockSpec((IPB, Cout, HW),
                               lambda p, i: (jnp.where(p == 1, i, 0), 0, 0)),
        scratch_shapes=[
            pltpu.VMEM((N, Cin, P), bf16),
            pltpu.VMEM((Cout, 128), f32),
            pltpu.VMEM((Cout, 128), f32),
            pltpu.VMEM((3 * Cout, 3 * Cout + Cin), bf16),
            pltpu.VMEM((Cout, 128), f32),
        ],
        compiler_params=pltpu.CompilerParams(
            dimension_semantics=("arbitrary", "arbitrary"),
            vmem_limit_bytes=64 * 1024 * 1024),
        cost_estimate=pl.CostEstimate(
            flops=2 * N * P * (3 * Cin * 3 * Cout + (3 * Cout + Cin) * 3 * Cout)
            + 2 * N * Cin * Cout * HW,
            transcendentals=0,
            bytes_accessed=4 * N * Cin * HW + 4 * N * Cout * HW),
    )(x_flat, w1s.astype(bf16), w2s.astype(bf16), wdm, bnmat)

    return out_flat.reshape(N, Cout, H, W)
